# Initial kernel scaffold; baseline (speedup 1.0000x reference)
import jax
import jax.numpy as jnp
from jax.experimental import pallas as pl


def kernel(user_ids, author_ids, author_tokens, age, user_table, author_table, text_table, age_mean, age_var):
    raise NotImplementedError("write your pallas kernel here")


# speedup vs baseline: 9.4893x; 9.4893x over previous
import functools

import jax
import jax.numpy as jnp
from jax import lax
from jax.experimental import pallas as pl
from jax.experimental.pallas import tpu as pltpu
from jax.experimental.pallas import tpu_sc as plsc

B = 16384
D = 32
L = 20
OUT_W = 3 * D + 1
G = 128


@functools.cache
def _build(nc: int, ns: int):
    nw = nc * ns
    bpw = B // nw
    ch = 64
    nchunk = bpw // ch
    rows_per_chunk = ch * L
    ng_text = rows_per_chunk // G
    ng_id = bpw // G
    tok_rows_w = bpw * L // G

    mesh = plsc.VectorSubcoreMesh(core_axis_name="c", subcore_axis_name="s")

    @functools.partial(
        pl.kernel,
        out_type=jax.ShapeDtypeStruct((B * OUT_W,), jnp.float32),
        mesh=mesh,
        scratch_types=[
            pltpu.VMEM((bpw,), jnp.int32),
            pltpu.VMEM((bpw,), jnp.int32),
            pltpu.VMEM((bpw * L,), jnp.int32),
            pltpu.VMEM((bpw,), jnp.float32),
            pltpu.VMEM((bpw, D), jnp.float32),
            pltpu.VMEM((bpw, D), jnp.float32),
            pltpu.VMEM((rows_per_chunk, D), jnp.float32),
            pltpu.VMEM((1, D), jnp.float32),
            pltpu.VMEM((2, 16), jnp.float32),
            pltpu.VMEM((bpw + 16,), jnp.float32),
            pltpu.VMEM((bpw + 16,), jnp.float32),
            pltpu.VMEM((ch * OUT_W,), jnp.float32),
            pltpu.SemaphoreType.DMA,
        ],
        compiler_params=pltpu.CompilerParams(needs_layout_passes=False, use_tc_tiling_on_sc=False),
    )
    def launch(uid_hbm, aid_hbm, tok_hbm, age_hbm, utab, atab, ttab,
               params_hbm, out_hbm, uid_v, aid_v, tok_v, age_v, u_rows,
               a_rows, trows, row0_v, params_v, inv_v, coef_v, out_c, sem):
        cid = lax.axis_index("c")
        sid = lax.axis_index("s")
        wid = cid * ns + sid
        base = wid * bpw

        pltpu.sync_copy(uid_hbm.at[pl.ds(base, bpw)], uid_v)
        pltpu.sync_copy(aid_hbm.at[pl.ds(base, bpw)], aid_v)
        pltpu.sync_copy(tok_hbm.at[pl.ds(base * L, bpw * L)], tok_v)
        pltpu.sync_copy(age_hbm.at[pl.ds(base, bpw)], age_v)
        pltpu.sync_copy(ttab.at[pl.ds(0, 1)], row0_v)
        pltpu.sync_copy(params_hbm, params_v)

        descs = []
        for k in range(ng_id):
            descs.append(pltpu.async_copy(
                utab.at[uid_v.at[pl.ds(k * G, G)]], u_rows.at[pl.ds(k * G, G)], sem))
            descs.append(pltpu.async_copy(
                atab.at[aid_v.at[pl.ds(k * G, G)]], a_rows.at[pl.ds(k * G, G)], sem))
        for dsc in descs:
            dsc.wait()

        iota16 = lax.iota(jnp.int32, 16)

        def cnt_body(k, carry):
            b0 = k * 16
            lane_b = iota16 + b0
            cnt = jnp.zeros((16,), jnp.float32)
            for j in range(L):
                flat = lane_b * L + j
                t = plsc.load_gather(tok_v, [flat])
                cnt = cnt + jnp.where(t != 0, jnp.float32(1.0), jnp.float32(0.0))
            inv_v[pl.ds(b0, 16)] = jnp.float32(1.0) / jnp.maximum(cnt, 1.0)
            coef_v[pl.ds(b0, 16)] = cnt - jnp.float32(L)
            return carry

        lax.fori_loop(0, bpw // 16, cnt_body, 0)

        r0a = row0_v[0, pl.ds(0, 16)]
        r0b = row0_v[0, pl.ds(16, 16)]
        mean_vec = params_v[0, pl.ds(0, 16)]
        scale_vec = params_v[1, pl.ds(0, 16)]

        for c in range(nchunk):
            tds = [pltpu.async_copy(
                       ttab.at[tok_v.at[pl.ds((c * ng_text + k) * G, G)]],
                       trows.at[pl.ds(k * G, G)], sem)
                   for k in range(ng_text)]
            for dsc in tds:
                dsc.wait()

            def b_body(bl, carry, c=c):
                b_abs = c * ch + bl
                r = bl * L
                acc0 = jnp.zeros((16,), jnp.float32)
                acc1 = jnp.zeros((16,), jnp.float32)
                for j in range(L):
                    acc0 = acc0 + trows[r + j, pl.ds(0, 16)]
                    acc1 = acc1 + trows[r + j, pl.ds(16, 16)]
                coef = coef_v[pl.ds(b_abs, 16)][0]
                inv = inv_v[pl.ds(b_abs, 16)][0]
                t0 = (acc0 + coef * r0a) * inv
                t1 = (acc1 + coef * r0b) * inv
                u0 = u_rows[b_abs, pl.ds(0, 16)]
                u1 = u_rows[b_abs, pl.ds(16, 16)]
                a0 = a_rows[b_abs, pl.ds(0, 16)]
                a1 = a_rows[b_abs, pl.ds(16, 16)]
                idx = iota16 + bl * OUT_W
                plsc.store_scatter(out_c, [idx], u0)
                plsc.store_scatter(out_c, [idx + 16], u1)
                plsc.store_scatter(out_c, [idx + 32], a0)
                plsc.store_scatter(out_c, [idx + 48], a1)
                plsc.store_scatter(out_c, [idx + 64], t0)
                plsc.store_scatter(out_c, [idx + 80], t1)
                return carry

            lax.fori_loop(0, ch, b_body, 0)

            for v in range(ch // 16):
                x = age_v[pl.ds(c * ch + v * 16, 16)]
                xn = (x - mean_vec) * scale_vec
                aidx = iota16 * OUT_W + (v * 16 * OUT_W + OUT_W - 1)
                plsc.store_scatter(out_c, [aidx], xn)

            pltpu.sync_copy(
                out_c, out_hbm.at[pl.ds((base + c * ch) * OUT_W, ch * OUT_W)])

    return launch


def kernel(user_ids, author_ids, author_tokens, age, user_table,
           author_table, text_table, age_mean, age_var):
    info = plsc.get_sparse_core_info()
    launch = _build(info.num_cores, info.num_subcores)
    tok_flat = author_tokens.reshape(-1)
    params = jnp.stack([
        jnp.full((16,), age_mean, jnp.float32),
        jnp.full((16,), lax.rsqrt(age_var), jnp.float32),
    ])
    flat = launch(user_ids, author_ids, tok_flat, age, user_table,
                  author_table, text_table, params)
    return flat.reshape(B, OUT_W)


# speedup vs baseline: 11.3402x; 1.1950x over previous
import functools

import jax
import jax.numpy as jnp
from jax import lax
from jax.experimental import pallas as pl
from jax.experimental.pallas import tpu as pltpu
from jax.experimental.pallas import tpu_sc as plsc

B = 16384
D = 32
L = 20
ROW_W = 128
OUT_W = 3 * D + 1
G = 128


@functools.cache
def _build(nc: int, ns: int):
    nw = nc * ns
    bpw = B // nw
    ch = 64
    nchunk = bpw // ch
    rows_per_chunk = ch * L
    ng_text = rows_per_chunk // G
    ng_id = bpw // G

    mesh = plsc.VectorSubcoreMesh(core_axis_name="c", subcore_axis_name="s")
    cparams = pltpu.CompilerParams(
        needs_layout_passes=False, use_tc_tiling_on_sc=False)

    @functools.partial(
        pl.kernel,
        out_type=jax.ShapeDtypeStruct((B * D,), jnp.float32),
        mesh=mesh,
        scratch_types=[
            pltpu.VMEM((bpw * L,), jnp.int32),
            pltpu.VMEM((rows_per_chunk, D), jnp.float32),
            pltpu.VMEM((rows_per_chunk, D), jnp.float32),
            pltpu.VMEM((1, D), jnp.float32),
            pltpu.VMEM((bpw + 16,), jnp.float32),
            pltpu.VMEM((bpw + 16,), jnp.float32),
            pltpu.VMEM((ch * D,), jnp.float32),
            pltpu.VMEM((ch * D,), jnp.float32),
            pltpu.SemaphoreType.DMA,
            pltpu.SemaphoreType.DMA,
        ],
        compiler_params=cparams,
    )
    def launch_text(tok_hbm, ttab, pooled_hbm, tok_v, tr0, tr1, row0_v,
                    inv_v, coef_v, pb0, pb1, sem_g, sem_o):
        cid = lax.axis_index("c")
        sid = lax.axis_index("s")
        wid = cid * ns + sid
        base = wid * bpw

        pltpu.sync_copy(tok_hbm.at[pl.ds(base * L, bpw * L)], tok_v)
        pltpu.sync_copy(ttab.at[pl.ds(0, 1)], row0_v)

        trs = (tr0, tr1)
        pbs = (pb0, pb1)

        def fire(c):
            return [pltpu.async_copy(
                        ttab.at[tok_v.at[pl.ds((c * ng_text + k) * G, G)]],
                        trs[c % 2].at[pl.ds(k * G, G)], sem_g)
                    for k in range(ng_text)]

        gds = fire(0)

        iota16 = lax.iota(jnp.int32, 16)

        def cnt_body(k, carry):
            b0 = k * 16
            lane_b = iota16 + b0
            cnt = jnp.zeros((16,), jnp.float32)
            for j in range(L):
                flat = lane_b * L + j
                t = plsc.load_gather(tok_v, [flat])
                cnt = cnt + jnp.where(t != 0, jnp.float32(1.0), jnp.float32(0.0))
            inv_v[pl.ds(b0, 16)] = jnp.float32(1.0) / jnp.maximum(cnt, 1.0)
            coef_v[pl.ds(b0, 16)] = cnt - jnp.float32(L)
            return carry

        lax.fori_loop(0, bpw // 16, cnt_body, 0)

        r0a = row0_v[0, pl.ds(0, 16)]
        r0b = row0_v[0, pl.ds(16, 16)]

        ods = {}
        for c in range(nchunk):
            nxt = fire(c + 1) if c + 1 < nchunk else []
            for dsc in gds:
                dsc.wait()
            gds = nxt
            if c >= 2:
                ods[c - 2].wait()
            tr = trs[c % 2]
            pb = pbs[c % 2]

            def b_body(bl, carry, tr=tr, pb=pb, c=c):
                b_abs = c * ch + bl
                r = bl * L
                acc0 = jnp.zeros((16,), jnp.float32)
                acc1 = jnp.zeros((16,), jnp.float32)
                for j in range(L):
                    acc0 = acc0 + tr[r + j, pl.ds(0, 16)]
                    acc1 = acc1 + tr[r + j, pl.ds(16, 16)]
                coef = coef_v[pl.ds(b_abs, 16)][0]
                inv = inv_v[pl.ds(b_abs, 16)][0]
                off = bl * D
                pb[pl.ds(off, 16)] = (acc0 + coef * r0a) * inv
                pb[pl.ds(off + 16, 16)] = (acc1 + coef * r0b) * inv
                return carry

            lax.fori_loop(0, ch, b_body, 0)
            ods[c] = pltpu.async_copy(
                pb, pooled_hbm.at[pl.ds((base + c * ch) * D, ch * D)], sem_o)

        for c in range(max(0, nchunk - 2), nchunk):
            ods[c].wait()

    @functools.partial(
        pl.kernel,
        out_type=jax.ShapeDtypeStruct((B * ROW_W,), jnp.float32),
        mesh=mesh,
        scratch_types=[
            pltpu.VMEM((bpw,), jnp.int32),
            pltpu.VMEM((bpw,), jnp.int32),
            pltpu.VMEM((bpw + 16,), jnp.float32),
            pltpu.VMEM((bpw, D), jnp.float32),
            pltpu.VMEM((bpw, D), jnp.float32),
            pltpu.VMEM((bpw * D,), jnp.float32),
            pltpu.VMEM((2, 16), jnp.float32),
            pltpu.VMEM((ch * ROW_W,), jnp.float32),
            pltpu.VMEM((ch * ROW_W,), jnp.float32),
            pltpu.SemaphoreType.DMA,
            pltpu.SemaphoreType.DMA,
        ],
        compiler_params=cparams,
    )
    def launch_asm(uid_hbm, aid_hbm, age_hbm, utab, atab, pooled_hbm,
                   params_hbm, out_hbm, uid_v, aid_v, age_v, u_rows, a_rows,
                   pool_v, params_v, ob0, ob1, sem_g, sem_o):
        cid = lax.axis_index("c")
        sid = lax.axis_index("s")
        wid = cid * ns + sid
        base = wid * bpw

        pltpu.sync_copy(uid_hbm.at[pl.ds(base, bpw)], uid_v)
        pltpu.sync_copy(aid_hbm.at[pl.ds(base, bpw)], aid_v)

        descs = []
        for k in range(ng_id):
            descs.append(pltpu.async_copy(
                utab.at[uid_v.at[pl.ds(k * G, G)]], u_rows.at[pl.ds(k * G, G)],
                sem_g))
            descs.append(pltpu.async_copy(
                atab.at[aid_v.at[pl.ds(k * G, G)]], a_rows.at[pl.ds(k * G, G)],
                sem_g))

        pltpu.sync_copy(age_hbm.at[pl.ds(base, bpw)], age_v.at[pl.ds(0, bpw)])
        pltpu.sync_copy(pooled_hbm.at[pl.ds(base * D, bpw * D)], pool_v)
        pltpu.sync_copy(params_hbm, params_v)

        for dsc in descs:
            dsc.wait()

        mean_vec = params_v[0, pl.ds(0, 16)]
        scale_vec = params_v[1, pl.ds(0, 16)]

        obs = (ob0, ob1)
        ods = {}
        for c in range(nchunk):
            if c >= 2:
                ods[c - 2].wait()
            ob = obs[c % 2]

            def b_body(bl, carry, ob=ob, c=c):
                b_abs = c * ch + bl
                off = bl * ROW_W
                ob[pl.ds(off, 16)] = u_rows[b_abs, pl.ds(0, 16)]
                ob[pl.ds(off + 16, 16)] = u_rows[b_abs, pl.ds(16, 16)]
                ob[pl.ds(off + 32, 16)] = a_rows[b_abs, pl.ds(0, 16)]
                ob[pl.ds(off + 48, 16)] = a_rows[b_abs, pl.ds(16, 16)]
                p = b_abs * D
                ob[pl.ds(off + 64, 16)] = pool_v[pl.ds(p, 16)]
                ob[pl.ds(off + 80, 16)] = pool_v[pl.ds(p + 16, 16)]
                agev = (age_v[pl.ds(b_abs, 16)] - mean_vec) * scale_vec
                ob[pl.ds(off + 96, 16)] = agev
                return carry

            lax.fori_loop(0, ch, b_body, 0)
            ods[c] = pltpu.async_copy(
                ob, out_hbm.at[pl.ds((base + c * ch) * ROW_W, ch * ROW_W)],
                sem_o)

        for c in range(max(0, nchunk - 2), nchunk):
            ods[c].wait()

    return launch_text, launch_asm


def kernel(user_ids, author_ids, author_tokens, age, user_table,
           author_table, text_table, age_mean, age_var):
    info = plsc.get_sparse_core_info()
    launch_text, launch_asm = _build(info.num_cores, info.num_subcores)
    pooled = launch_text(author_tokens.reshape(-1), text_table)
    params = jnp.stack([
        jnp.full((16,), age_mean, jnp.float32),
        jnp.full((16,), lax.rsqrt(age_var), jnp.float32),
    ])
    flat = launch_asm(user_ids, author_ids, age, user_table, author_table,
                      pooled, params)
    return flat.reshape(B, ROW_W)[:, :OUT_W]


# speedup vs baseline: 11.3657x; 1.0023x over previous
import functools

import jax
import jax.numpy as jnp
from jax import lax
from jax.experimental import pallas as pl
from jax.experimental.pallas import tpu as pltpu
from jax.experimental.pallas import tpu_sc as plsc

B = 16384
D = 32
L = 20
ROW_W = 128
OUT_W = 3 * D + 1
G = 128


@functools.cache
def _build(nc: int, ns: int):
    nw = nc * ns
    bpw = B // nw
    ch = 64
    nchunk = bpw // ch
    rows_per_chunk = ch * L
    ng_text = rows_per_chunk // G
    ng_id = bpw // G

    mesh = plsc.VectorSubcoreMesh(core_axis_name="c", subcore_axis_name="s")
    cparams = pltpu.CompilerParams(
        needs_layout_passes=False, use_tc_tiling_on_sc=False)

    @functools.partial(
        pl.kernel,
        out_type=jax.ShapeDtypeStruct((B * D,), jnp.float32),
        mesh=mesh,
        scratch_types=[
            pltpu.VMEM((bpw * L,), jnp.int32),
            pltpu.VMEM((rows_per_chunk, D), jnp.float32),
            pltpu.VMEM((rows_per_chunk, D), jnp.float32),
            pltpu.VMEM((1, D), jnp.float32),
            pltpu.VMEM((bpw + 16,), jnp.float32),
            pltpu.VMEM((bpw + 16,), jnp.float32),
            pltpu.VMEM((ch * D,), jnp.float32),
            pltpu.VMEM((ch * D,), jnp.float32),
            pltpu.SemaphoreType.DMA,
            pltpu.SemaphoreType.DMA,
        ],
        compiler_params=cparams,
    )
    def launch_text(tok_hbm, ttab, pooled_hbm, tok_v, tr0, tr1, row0_v,
                    inv_v, coef_v, pb0, pb1, sem_g, sem_o):
        cid = lax.axis_index("c")
        sid = lax.axis_index("s")
        wid = cid * ns + sid
        base = wid * bpw

        pltpu.sync_copy(tok_hbm.at[pl.ds(base * L, bpw * L)], tok_v)
        pltpu.sync_copy(ttab.at[pl.ds(0, 1)], row0_v)

        trs = (tr0, tr1)
        pbs = (pb0, pb1)

        def fire(c):
            return [pltpu.async_copy(
                        ttab.at[tok_v.at[pl.ds((c * ng_text + k) * G, G)]],
                        trs[c % 2].at[pl.ds(k * G, G)], sem_g)
                    for k in range(ng_text)]

        gds = fire(0)

        iota16 = lax.iota(jnp.int32, 16)

        def cnt_body(k, carry):
            b0 = k * 16
            lane_b = iota16 + b0
            cnt = jnp.zeros((16,), jnp.float32)
            for j in range(L):
                flat = lane_b * L + j
                t = plsc.load_gather(tok_v, [flat])
                cnt = cnt + jnp.where(t != 0, jnp.float32(1.0), jnp.float32(0.0))
            inv_v[pl.ds(b0, 16)] = jnp.float32(1.0) / jnp.maximum(cnt, 1.0)
            coef_v[pl.ds(b0, 16)] = cnt - jnp.float32(L)
            return carry

        lax.fori_loop(0, bpw // 16, cnt_body, 0)

        r0a = row0_v[0, pl.ds(0, 16)]
        r0b = row0_v[0, pl.ds(16, 16)]

        ods = {}
        for c in range(nchunk):
            nxt = fire(c + 1) if c + 1 < nchunk else []
            for dsc in gds:
                dsc.wait()
            gds = nxt
            if c >= 2:
                ods[c - 2].wait()
            tr = trs[c % 2]
            pb = pbs[c % 2]

            def b_body(bl, carry, tr=tr, pb=pb, c=c):
                b_abs = c * ch + bl
                r = bl * L
                acc0 = jnp.zeros((16,), jnp.float32)
                acc1 = jnp.zeros((16,), jnp.float32)
                for j in range(L):
                    acc0 = acc0 + tr[r + j, pl.ds(0, 16)]
                    acc1 = acc1 + tr[r + j, pl.ds(16, 16)]
                coef = coef_v[pl.ds(b_abs, 16)][0]
                inv = inv_v[pl.ds(b_abs, 16)][0]
                off = bl * D
                pb[pl.ds(off, 16)] = (acc0 + coef * r0a) * inv
                pb[pl.ds(off + 16, 16)] = (acc1 + coef * r0b) * inv
                return carry

            lax.fori_loop(0, ch, b_body, 0)
            ods[c] = pltpu.async_copy(
                pb, pooled_hbm.at[pl.ds((base + c * ch) * D, ch * D)], sem_o)

        for c in range(max(0, nchunk - 2), nchunk):
            ods[c].wait()

    @functools.partial(
        pl.kernel,
        out_type=jax.ShapeDtypeStruct((B, ROW_W), jnp.float32),
        mesh=mesh,
        scratch_types=[
            pltpu.VMEM((bpw,), jnp.int32),
            pltpu.VMEM((bpw,), jnp.int32),
            pltpu.VMEM((bpw + 16,), jnp.float32),
            pltpu.VMEM((bpw, D), jnp.float32),
            pltpu.VMEM((bpw, D), jnp.float32),
            pltpu.VMEM((bpw * D,), jnp.float32),
            pltpu.VMEM((2 * 16,), jnp.float32),
            pltpu.VMEM((ch, ROW_W), jnp.float32),
            pltpu.VMEM((ch, ROW_W), jnp.float32),
            pltpu.SemaphoreType.DMA,
            pltpu.SemaphoreType.DMA,
        ],
        compiler_params=cparams,
    )
    def launch_asm(uid_hbm, aid_hbm, age_hbm, utab, atab, pooled_hbm,
                   params_hbm, out_hbm, uid_v, aid_v, age_v, u_rows, a_rows,
                   pool_v, params_v, ob0, ob1, sem_g, sem_o):
        cid = lax.axis_index("c")
        sid = lax.axis_index("s")
        wid = cid * ns + sid
        base = wid * bpw

        pltpu.sync_copy(uid_hbm.at[pl.ds(base, bpw)], uid_v)
        pltpu.sync_copy(aid_hbm.at[pl.ds(base, bpw)], aid_v)

        descs = []
        for k in range(ng_id):
            descs.append(pltpu.async_copy(
                utab.at[uid_v.at[pl.ds(k * G, G)]], u_rows.at[pl.ds(k * G, G)],
                sem_g))
            descs.append(pltpu.async_copy(
                atab.at[aid_v.at[pl.ds(k * G, G)]], a_rows.at[pl.ds(k * G, G)],
                sem_g))

        pltpu.sync_copy(age_hbm.at[pl.ds(base, bpw)], age_v.at[pl.ds(0, bpw)])
        pltpu.sync_copy(pooled_hbm.at[pl.ds(base * D, bpw * D)], pool_v)
        pltpu.sync_copy(params_hbm, params_v)

        for dsc in descs:
            dsc.wait()

        mean_vec = params_v[pl.ds(0, 16)]
        scale_vec = params_v[pl.ds(16, 16)]

        obs = (ob0, ob1)
        ods = {}
        for c in range(nchunk):
            if c >= 2:
                ods[c - 2].wait()
            ob = obs[c % 2]

            def b_body(bl, carry, ob=ob, c=c):
                b_abs = c * ch + bl
                ob[bl, pl.ds(0, 16)] = u_rows[b_abs, pl.ds(0, 16)]
                ob[bl, pl.ds(16, 16)] = u_rows[b_abs, pl.ds(16, 16)]
                ob[bl, pl.ds(32, 16)] = a_rows[b_abs, pl.ds(0, 16)]
                ob[bl, pl.ds(48, 16)] = a_rows[b_abs, pl.ds(16, 16)]
                p = b_abs * D
                ob[bl, pl.ds(64, 16)] = pool_v[pl.ds(p, 16)]
                ob[bl, pl.ds(80, 16)] = pool_v[pl.ds(p + 16, 16)]
                agev = (age_v[pl.ds(b_abs, 16)] - mean_vec) * scale_vec
                ob[bl, pl.ds(96, 16)] = agev
                return carry

            lax.fori_loop(0, ch, b_body, 0)
            ods[c] = pltpu.async_copy(
                ob, out_hbm.at[pl.ds(base + c * ch, ch), :], sem_o)

        for c in range(max(0, nchunk - 2), nchunk):
            ods[c].wait()

    return launch_text, launch_asm


def kernel(user_ids, author_ids, author_tokens, age, user_table,
           author_table, text_table, age_mean, age_var):
    info = plsc.get_sparse_core_info()
    launch_text, launch_asm = _build(info.num_cores, info.num_subcores)
    pooled = launch_text(author_tokens.reshape(-1), text_table)
    params = jnp.concatenate([
        jnp.full((16,), age_mean, jnp.float32),
        jnp.full((16,), lax.rsqrt(age_var), jnp.float32),
    ])
    out = launch_asm(user_ids, author_ids, age, user_table, author_table,
                     pooled, params)
    return out[:, :OUT_W]
